# baseline (speedup 1.0000x reference)
import functools

import jax
import jax.numpy as jnp
from jax import lax
from jax.experimental import pallas as pl
from jax.experimental.pallas import tpu as pltpu
from jax.experimental.pallas import tpu_sc as plsc

_N = 7
_M = _N * (_N - 1) // 2


def _body(x_hbm, vec_hbm, out_hbm, x_s, buf_s, res_s):
    pltpu.sync_copy(x_hbm, x_s)
    xs = [x_s[i] for i in range(_N)]
    idx = jnp.int32(0)
    k = 0
    for i in range(1, _N):
        for j in range(i):
            idx = idx + jnp.where(xs[i] == xs[j], jnp.int32(1 << k), jnp.int32(0))
            k += 1
    pltpu.sync_copy(vec_hbm.at[idx >> 7], buf_s)
    res_s[0] = buf_s[idx & jnp.int32(127)]
    pltpu.sync_copy(res_s, out_hbm)


@jax.jit
def kernel(x, vec):
    mesh = plsc.ScalarSubcoreMesh(axis_name="c", num_cores=1)
    run = functools.partial(
        pl.kernel,
        mesh=mesh,
        out_type=jax.ShapeDtypeStruct((8,), jnp.float32),
        scratch_types=[
            pltpu.SMEM((8,), jnp.int32),
            pltpu.SMEM((128,), jnp.float32),
            pltpu.SMEM((8,), jnp.float32),
        ],
        compiler_params=pltpu.CompilerParams(needs_layout_passes=False),
    )(_body)
    xp = jnp.zeros((8,), jnp.int32).at[:_N].set(x.astype(jnp.int32))
    out = run(xp, vec.reshape(2 ** _M // 128, 128))
    return out[0]
